# Initial kernel scaffold; baseline (speedup 1.0000x reference)
import jax
import jax.numpy as jnp
from jax.experimental import pallas as pl


def kernel(keys, values, attention_scores, timestamps):
    raise NotImplementedError("write your pallas kernel here")


# speedup vs baseline: 2.0116x; 2.0116x over previous
import functools

import jax
import jax.numpy as jnp
from jax import lax
from jax.experimental import pallas as pl
from jax.experimental.pallas import tpu as pltpu

_CACHE_SIZE = 4096
_HEAVY_RATIO = 0.1
_RECENT_RATIO = 0.1

_LANES = 128


def _avg_floor(a, b):
    return (a & b) + ((a ^ b) >> 1)


def _orderable(x_f32):
    i = lax.bitcast_convert_type(x_f32, jnp.int32)
    return jnp.where(i < 0, i ^ jnp.int32(0x7FFFFFFF), i)


def _selection_scalars(u, gidx, k):
    n = u.size

    def vbody(_, carry):
        lo, hi = carry
        mid = _avg_floor(lo, hi) + ((lo ^ hi) & 1)
        cnt = jnp.sum((u >= mid).astype(jnp.int32))
        go = lo < hi
        take = cnt >= k
        new_lo = jnp.where(take, mid, lo)
        new_hi = jnp.where(take, hi, mid - 1)
        return (jnp.where(go, new_lo, lo), jnp.where(go, new_hi, hi))

    int_min = jnp.int32(-2147483648)
    int_max = jnp.int32(2147483647)
    lo, hi = lax.fori_loop(0, 32, vbody, (int_min, int_max))
    T = lo

    eq = u == T
    cnt_gt = jnp.sum((u > T).astype(jnp.int32))
    ne = k - cnt_gt

    def ibody(_, carry):
        lo, hi = carry
        mid = _avg_floor(lo, hi)
        cnt = jnp.sum((eq & (gidx <= mid)).astype(jnp.int32))
        go = lo < hi
        take = cnt >= ne
        new_lo = jnp.where(take, lo, mid + 1)
        new_hi = jnp.where(take, mid, hi)
        return (jnp.where(go, new_lo, lo), jnp.where(go, new_hi, hi))

    lo2, hi2 = lax.fori_loop(0, 14, ibody, (jnp.int32(0), jnp.int32(n - 1)))
    return T, lo2


def _fused_kernel(scores2d_ref, ts2d_ref, scol_ref, tcol_ref,
                  keys_ref, values_ref, out_ref, sel_ref, *, k, block_rows):
    i = pl.program_id(0)

    @pl.when(i == 0)
    def _():
        n = scores2d_ref.shape[0] * scores2d_ref.shape[1]
        rows = scores2d_ref.shape[0]
        ri = lax.broadcasted_iota(jnp.int32, (rows, _LANES), 0)
        ci = lax.broadcasted_iota(jnp.int32, (rows, _LANES), 1)
        gidx = ri * _LANES + ci
        us = _orderable(scores2d_ref[...])
        ut = _orderable(ts2d_ref[...])
        Ts, Ps = _selection_scalars(us, gidx, k)
        Tt, Pt = _selection_scalars(ut, gidx, k)
        sel_ref[0] = Ts
        sel_ref[1] = Ps
        sel_ref[2] = Tt
        sel_ref[3] = Pt

    ridx = i * block_rows + lax.broadcasted_iota(jnp.int32, (block_rows, 1), 0)
    us = _orderable(scol_ref[...])
    ut = _orderable(tcol_ref[...])
    Ts, Ps, Tt, Pt = sel_ref[0], sel_ref[1], sel_ref[2], sel_ref[3]
    keep_s = (us > Ts) | ((us == Ts) & (ridx <= Ps))
    keep_t = (ut > Tt) | ((ut == Tt) & (ridx <= Pt))
    m = (keep_s | keep_t).astype(jnp.float32)
    out_ref[0, :, :] = keys_ref[...] * m
    out_ref[1, :, :] = values_ref[...] * m


@jax.jit
def kernel(keys, values, attention_scores, timestamps):
    cache_len, hidden = keys.shape
    if cache_len <= _CACHE_SIZE:
        return jnp.stack([keys, values], axis=0)

    k = max(1, int(cache_len * _HEAVY_RATIO))

    block_rows = 512
    nb = cache_len // block_rows
    rows2d = cache_len // _LANES

    scores2d = attention_scores.reshape(rows2d, _LANES)
    ts2d = timestamps.reshape(rows2d, _LANES)
    scol = attention_scores.reshape(cache_len, 1)
    tcol = timestamps.reshape(cache_len, 1)

    in_specs = [
            pl.BlockSpec((rows2d, _LANES), lambda i: (0, 0)),
            pl.BlockSpec((rows2d, _LANES), lambda i: (0, 0)),
            pl.BlockSpec((block_rows, 1), lambda i: (i, 0)),
            pl.BlockSpec((block_rows, 1), lambda i: (i, 0)),
            pl.BlockSpec((block_rows, hidden), lambda i: (i, 0)),
            pl.BlockSpec((block_rows, hidden), lambda i: (i, 0)),
        ]

    return pl.pallas_call(
        functools.partial(_fused_kernel, k=k, block_rows=block_rows),
        grid=(nb,),
        in_specs=in_specs,
        out_specs=pl.BlockSpec((2, block_rows, hidden), lambda i: (0, i, 0)),
        out_shape=jax.ShapeDtypeStruct((2, cache_len, hidden), jnp.float32),
        scratch_shapes=[pltpu.SMEM((4,), jnp.int32)],
    )(scores2d, ts2d, scol, tcol, keys, values)
